# baseline (speedup 1.0000x reference)
import functools

import jax
import jax.numpy as jnp
from jax import lax
from jax.experimental import pallas as pl
from jax.experimental.pallas import tpu as pltpu
from jax.experimental.pallas import tpu_sc as plsc

_BM = 1024
_BN = 1024


def _argmin_body(f_ref, w_ref, f2_ref, w2_ref, idx_out, diff_out,
                 best_val, best_idx):
    m = pl.program_id(0)
    n = pl.program_id(1)
    dot = lax.dot_general(
        f_ref[...].astype(jnp.bfloat16), w_ref[...].astype(jnp.bfloat16),
        (((1,), (1,)), ((), ())),
        preferred_element_type=jnp.float32)
    s = (f2_ref[...] + w2_ref[...]) - 2.0 * dot
    mn = jnp.min(s, axis=1, keepdims=True)
    lane = lax.broadcasted_iota(jnp.int32, s.shape, 1)
    arg = jnp.min(jnp.where(s == mn, lane, _BN), axis=1, keepdims=True)
    arg = arg + n * _BN

    @pl.when(n == 0)
    def _init():
        best_val[...] = mn
        best_idx[...] = arg

    @pl.when(n > 0)
    def _update():
        pred = mn < best_val[...]
        best_val[...] = jnp.where(pred, mn, best_val[...])
        best_idx[...] = jnp.where(pred, arg, best_idx[...])

    @pl.when(n == pl.num_programs(1) - 1)
    def _finish():
        idx_out[...] = best_idx[...]
        part = jnp.sum(best_val[...], keepdims=True)

        @pl.when(m == 0)
        def _first():
            diff_out[...] = part

        @pl.when(m > 0)
        def _acc():
            diff_out[...] = diff_out[...] + part


def _distance_argmin(flat, W, f2, w2):
    ntok, dim = flat.shape
    ncode = W.shape[0]
    grid = (ntok // _BM, ncode // _BN)
    return pl.pallas_call(
        _argmin_body,
        grid=grid,
        in_specs=[
            pl.BlockSpec((_BM, dim), lambda m, n: (m, 0)),
            pl.BlockSpec((_BN, dim), lambda m, n: (n, 0)),
            pl.BlockSpec((_BM, 1), lambda m, n: (m, 0)),
            pl.BlockSpec((1, _BN), lambda m, n: (0, n)),
        ],
        out_specs=[
            pl.BlockSpec((_BM, 1), lambda m, n: (m, 0)),
            pl.BlockSpec((1, 1), lambda m, n: (0, 0)),
        ],
        out_shape=[
            jax.ShapeDtypeStruct((ntok, 1), jnp.int32),
            jax.ShapeDtypeStruct((1, 1), jnp.float32),
        ],
        scratch_shapes=[
            pltpu.VMEM((_BM, 1), jnp.float32),
            pltpu.VMEM((_BM, 1), jnp.int32),
        ],
    )(flat, W, f2, w2)


@functools.cache
def _make_sc_gather(ntok, dim):
    info = plsc.get_sparse_core_info()
    nw = info.num_cores * info.num_subcores
    b_per_w = ntok // nw
    mesh = plsc.VectorSubcoreMesh(core_axis_name="c", subcore_axis_name="s")

    @functools.partial(
        pl.kernel, mesh=mesh,
        out_type=jax.ShapeDtypeStruct((ntok, dim), jnp.float32),
        scratch_types=[
            pltpu.VMEM((b_per_w,), jnp.int32),
            pltpu.VMEM((b_per_w, dim), jnp.float32),
            pltpu.SemaphoreType.DMA,
        ],
    )
    def gather(table_hbm, idx_hbm, out_hbm, idx_v, rows_v, sem):
        wid = lax.axis_index("s") * info.num_cores + lax.axis_index("c")
        base = wid * b_per_w
        pltpu.sync_copy(idx_hbm.at[pl.ds(base, b_per_w)], idx_v)
        pltpu.async_copy(table_hbm.at[idx_v], rows_v, sem).wait()
        pltpu.sync_copy(rows_v, out_hbm.at[pl.ds(base, b_per_w)])

    return gather


def kernel(z, W):
    dim = W.shape[1]
    flat = z.reshape(-1, dim)
    f2 = jnp.sum(flat * flat, axis=1, keepdims=True)
    w2 = jnp.sum(W * W, axis=1)[None, :]
    idx2d, diff_sum = _distance_argmin(flat, W, f2, w2)
    idxs = idx2d[:, 0]
    z_q = _make_sc_gather(flat.shape[0], dim)(W, idxs)
    z_q = z_q.reshape(z.shape)
    v = diff_sum[0, 0] / (flat.shape[0] * dim)
    diff = v + v
    z_q_st = z + lax.stop_gradient(z_q - z)
    return (z_q_st, idxs.reshape(z.shape[:-1]), diff)


# speedup vs baseline: 1.3279x; 1.3279x over previous
import functools

import jax
import jax.numpy as jnp
from jax import lax
from jax.experimental import pallas as pl
from jax.experimental.pallas import tpu as pltpu
from jax.experimental.pallas import tpu_sc as plsc

_BM = 1024
_BN = 8192


def _argmin_body(f_ref, w_ref, f2_ref, w2_ref, idx_out, diff_out,
                 best_val, best_idx):
    m = pl.program_id(0)
    n = pl.program_id(1)
    dot = lax.dot_general(
        f_ref[...], w_ref[...], (((1,), (1,)), ((), ())),
        preferred_element_type=jnp.float32)
    s = (f2_ref[...] + w2_ref[...]) - 2.0 * dot
    mn = jnp.min(s, axis=1, keepdims=True)
    lane = lax.broadcasted_iota(jnp.int32, s.shape, 1)
    arg = jnp.min(jnp.where(s == mn, lane, _BN), axis=1, keepdims=True)
    arg = arg + n * _BN

    @pl.when(n == 0)
    def _init():
        best_val[...] = mn
        best_idx[...] = arg

    @pl.when(n > 0)
    def _update():
        pred = mn < best_val[...]
        best_val[...] = jnp.where(pred, mn, best_val[...])
        best_idx[...] = jnp.where(pred, arg, best_idx[...])

    @pl.when(n == pl.num_programs(1) - 1)
    def _finish():
        idx_out[...] = best_idx[...]
        part = jnp.sum(best_val[...], keepdims=True)

        @pl.when(m == 0)
        def _first():
            diff_out[...] = part

        @pl.when(m > 0)
        def _acc():
            diff_out[...] = diff_out[...] + part


def _distance_argmin(flat, W, f2, w2):
    ntok, dim = flat.shape
    ncode = W.shape[0]
    grid = (ntok // _BM, ncode // _BN)
    return pl.pallas_call(
        _argmin_body,
        grid=grid,
        in_specs=[
            pl.BlockSpec((_BM, dim), lambda m, n: (m, 0)),
            pl.BlockSpec((_BN, dim), lambda m, n: (n, 0)),
            pl.BlockSpec((_BM, 1), lambda m, n: (m, 0)),
            pl.BlockSpec((1, _BN), lambda m, n: (0, n)),
        ],
        out_specs=[
            pl.BlockSpec((_BM, 1), lambda m, n: (m, 0)),
            pl.BlockSpec((1, 1), lambda m, n: (0, 0)),
        ],
        out_shape=[
            jax.ShapeDtypeStruct((ntok, 1), jnp.int32),
            jax.ShapeDtypeStruct((1, 1), jnp.float32),
        ],
        scratch_shapes=[
            pltpu.VMEM((_BM, 1), jnp.float32),
            pltpu.VMEM((_BM, 1), jnp.int32),
        ],
    )(flat, W, f2, w2)


@functools.cache
def _make_sc_gather(ntok, dim):
    info = plsc.get_sparse_core_info()
    nw = info.num_cores * info.num_subcores
    b_per_w = ntok // nw
    mesh = plsc.VectorSubcoreMesh(core_axis_name="c", subcore_axis_name="s")

    @functools.partial(
        pl.kernel, mesh=mesh,
        out_type=jax.ShapeDtypeStruct((ntok, dim), jnp.float32),
        scratch_types=[
            pltpu.VMEM((b_per_w,), jnp.int32),
            pltpu.VMEM((b_per_w, dim), jnp.float32),
            pltpu.SemaphoreType.DMA,
        ],
    )
    def gather(table_hbm, idx_hbm, out_hbm, idx_v, rows_v, sem):
        wid = lax.axis_index("s") * info.num_cores + lax.axis_index("c")
        base = wid * b_per_w
        pltpu.sync_copy(idx_hbm.at[pl.ds(base, b_per_w)], idx_v)
        pltpu.async_copy(table_hbm.at[idx_v], rows_v, sem).wait()
        pltpu.sync_copy(rows_v, out_hbm.at[pl.ds(base, b_per_w)])

    return gather


def kernel(z, W):
    dim = W.shape[1]
    flat = z.reshape(-1, dim)
    f2 = jnp.sum(flat * flat, axis=1, keepdims=True)
    w2 = jnp.sum(W * W, axis=1)[None, :]
    idx2d, diff_sum = _distance_argmin(flat, W, f2, w2)
    idxs = idx2d[:, 0]
    z_q = _make_sc_gather(flat.shape[0], dim)(W, idxs)
    z_q = z_q.reshape(z.shape)
    v = diff_sum[0, 0] / (flat.shape[0] * dim)
    diff = v + v
    z_q_st = z + lax.stop_gradient(z_q - z)
    return (z_q_st, idxs.reshape(z.shape[:-1]), diff)
